# baseline (speedup 1.0000x reference)
import functools

import jax
import jax.numpy as jnp
from jax import lax
from jax.experimental import pallas as pl
from jax.experimental.pallas import tpu as pltpu
from jax.experimental.pallas import tpu_sc as plsc

_N_USER = 10000
_N_ITEM = 10000
_D = 128
_E = 320000

_NUM_TILES = 16
_CHUNK = 80
_EPT = 20480
_NCHUNKS = _EPT // _CHUNK
_EPAD = _EPT * _NUM_TILES
_G = 8
_NPAIRS = _NCHUNKS // 2
_N_PAD = 10240
_ROWS_PER_TILE = _N_PAD // _NUM_TILES


def _sc_aggregate(table, src_flat, dst_flat, zeros):
    mesh = plsc.VectorSubcoreMesh(core_axis_name="c", subcore_axis_name="s")

    @functools.partial(
        pl.kernel,
        out_type=jax.ShapeDtypeStruct((2, _N_PAD, _D), jnp.float32),
        mesh=mesh,
        scratch_types=[
            pltpu.VMEM((_CHUNK,), jnp.int32),
            pltpu.VMEM((_CHUNK,), jnp.int32),
            pltpu.VMEM((_CHUNK, _D), jnp.float32),
            pltpu.VMEM_SHARED((_N_PAD, _D), jnp.float32),
        ],
    )
    def agg_kernel(table_hbm, src_hbm, dst_hbm, zeros_hbm, out_hbm,
                   src_v, dst_v, rows_v, acc_sh):
        c = lax.axis_index("c")
        s = lax.axis_index("s")
        rbase = s * _ROWS_PER_TILE
        ebase = c * _EPAD + s * _EPT

        pltpu.sync_copy(zeros_hbm.at[pl.ds(rbase, _ROWS_PER_TILE)],
                        acc_sh.at[pl.ds(rbase, _ROWS_PER_TILE)])
        plsc.subcore_barrier()

        @pl.loop(0, _NCHUNKS)
        def _(i):
            e0 = ebase + i * _CHUNK
            pltpu.sync_copy(src_hbm.at[pl.ds(e0, _CHUNK)], src_v)
            pltpu.sync_copy(dst_hbm.at[pl.ds(e0, _CHUNK)], dst_v)
            pltpu.sync_copy(table_hbm.at[src_v], rows_v)
            pltpu.sync_copy(rows_v, acc_sh.at[dst_v], add=True)

        plsc.subcore_barrier()
        pltpu.sync_copy(acc_sh.at[pl.ds(rbase, _ROWS_PER_TILE)],
                        out_hbm.at[c, pl.ds(rbase, _ROWS_PER_TILE)])

    return agg_kernel(table, src_flat, dst_flat, zeros)


def _affine_kernel(agg0_ref, agg1_ref, xi_ref, xu_ref,
                   wm0_ref, wr0_ref, b0_ref, wm1_ref, wr1_ref, b1_ref,
                   oi_ref, ou_ref):
    oi_ref[...] = (
        jnp.dot(agg0_ref[0], wm0_ref[...], preferred_element_type=jnp.float32)
        + jnp.dot(xi_ref[...], wr0_ref[...], preferred_element_type=jnp.float32)
        + b0_ref[...]
    )
    ou_ref[...] = (
        jnp.dot(agg1_ref[0], wm1_ref[...], preferred_element_type=jnp.float32)
        + jnp.dot(xu_ref[...], wr1_ref[...], preferred_element_type=jnp.float32)
        + b1_ref[...]
    )


def _tc_epilogue(agg, x_item, x_user, wm0, wr0, b0, wm1, wr1, b1):
    n = x_item.shape[0]
    blk = 2000
    grid = (n // blk,)
    row_spec = pl.BlockSpec((blk, _D), lambda i: (i, 0))
    w_spec = pl.BlockSpec((_D, _D), lambda i: (0, 0))
    b_spec = pl.BlockSpec((1, _D), lambda i: (0, 0))
    return pl.pallas_call(
        _affine_kernel,
        grid=grid,
        in_specs=[
            pl.BlockSpec((1, blk, _D), lambda i: (0, i, 0)),
            pl.BlockSpec((1, blk, _D), lambda i: (1, i, 0)),
            row_spec, row_spec,
            w_spec, w_spec, b_spec,
            w_spec, w_spec, b_spec,
        ],
        out_specs=[row_spec, row_spec],
        out_shape=[
            jax.ShapeDtypeStruct((n, _D), jnp.float32),
            jax.ShapeDtypeStruct((n, _D), jnp.float32),
        ],
    )(agg, agg, x_item, x_user, wm0, wr0, b0.reshape(1, _D),
      wm1, wr1, b1.reshape(1, _D))


def kernel(x_user, x_item, edge_index_u2i, edge_index_i2u,
           W_msg_u2i, W_root_u2i, b_u2i,
           W_msg_i2u, W_root_i2u, b_i2u):
    table = jnp.concatenate([x_user, x_item], axis=0)
    npad_e = _EPAD - _E
    src_pad = jnp.zeros((npad_e,), jnp.int32)
    dst_pad = jnp.full((npad_e,), _N_PAD - 1, jnp.int32)
    src_flat = jnp.concatenate([
        edge_index_u2i[0].astype(jnp.int32), src_pad,
        edge_index_i2u[0].astype(jnp.int32) + _N_USER, src_pad,
    ])
    dst_flat = jnp.concatenate([
        edge_index_u2i[1].astype(jnp.int32), dst_pad,
        edge_index_i2u[1].astype(jnp.int32), dst_pad,
    ])
    zeros = jnp.zeros((_N_PAD, _D), jnp.float32)
    agg = _sc_aggregate(table, src_flat, dst_flat, zeros)
    out_item, out_user = _tc_epilogue(
        agg, x_item, x_user,
        W_msg_u2i, W_root_u2i, b_u2i,
        W_msg_i2u, W_root_i2u, b_i2u)
    return (out_user, out_item)


# speedup vs baseline: 1.5648x; 1.5648x over previous
import functools

import jax
import jax.numpy as jnp
from jax import lax
from jax.experimental import pallas as pl
from jax.experimental.pallas import tpu as pltpu
from jax.experimental.pallas import tpu_sc as plsc

_N_USER = 10000
_N_ITEM = 10000
_D = 128
_E = 320000

_NUM_TILES = 16
_CHUNK = 80
_EPT = 20480
_NCHUNKS = _EPT // _CHUNK
_EPAD = _EPT * _NUM_TILES
_G = 8
_NPAIRS = _NCHUNKS // 2
_N_PAD = 10240
_ROWS_PER_TILE = _N_PAD // _NUM_TILES


def _sc_aggregate(table, src_flat, dst_flat, zeros):
    mesh = plsc.VectorSubcoreMesh(core_axis_name="c", subcore_axis_name="s")

    @functools.partial(
        pl.kernel,
        out_type=jax.ShapeDtypeStruct((2, _N_PAD, _D), jnp.float32),
        mesh=mesh,
        scratch_types=[
            pltpu.VMEM((_CHUNK,), jnp.int32),
            pltpu.VMEM((_CHUNK,), jnp.int32),
            pltpu.VMEM((_CHUNK, _D), jnp.float32),
            pltpu.VMEM_SHARED((_N_PAD, _D), jnp.float32),
        ],
    )
    def agg_kernel(table_hbm, src_hbm, dst_hbm, zeros_hbm, out_hbm,
                   src_v, dst_v, rows_v, acc_sh):
        c = lax.axis_index("c")
        s = lax.axis_index("s")
        rbase = s * _ROWS_PER_TILE
        ebase = c * _EPAD + s * _EPT

        pltpu.sync_copy(zeros_hbm.at[pl.ds(rbase, _ROWS_PER_TILE)],
                        acc_sh.at[pl.ds(rbase, _ROWS_PER_TILE)])
        plsc.subcore_barrier()

        @pl.loop(0, _NCHUNKS)
        def _(i):
            e0 = ebase + i * _CHUNK
            pltpu.sync_copy(src_hbm.at[pl.ds(e0, _CHUNK)], src_v)
            pltpu.sync_copy(dst_hbm.at[pl.ds(e0, _CHUNK)], dst_v)
            pltpu.sync_copy(table_hbm.at[src_v], rows_v)
            pltpu.sync_copy(rows_v, acc_sh.at[dst_v], add=True)

        plsc.subcore_barrier()
        pltpu.sync_copy(acc_sh.at[pl.ds(rbase, _ROWS_PER_TILE)],
                        out_hbm.at[c, pl.ds(rbase, _ROWS_PER_TILE)])

    return agg_kernel(table, src_flat, dst_flat, zeros)


def _affine_kernel(agg0_ref, agg1_ref, xi_ref, xu_ref,
                   wm0_ref, wr0_ref, b0_ref, wm1_ref, wr1_ref, b1_ref,
                   oi_ref, ou_ref):
    oi_ref[...] = (
        jnp.dot(agg0_ref[0], wm0_ref[...], preferred_element_type=jnp.float32)
        + jnp.dot(xi_ref[...], wr0_ref[...], preferred_element_type=jnp.float32)
        + b0_ref[...]
    )
    ou_ref[...] = (
        jnp.dot(agg1_ref[0], wm1_ref[...], preferred_element_type=jnp.float32)
        + jnp.dot(xu_ref[...], wr1_ref[...], preferred_element_type=jnp.float32)
        + b1_ref[...]
    )


def _tc_epilogue(agg, x_item, x_user, wm0, wr0, b0, wm1, wr1, b1):
    n = x_item.shape[0]
    blk = 2000
    grid = (n // blk,)
    row_spec = pl.BlockSpec((blk, _D), lambda i: (i, 0))
    w_spec = pl.BlockSpec((_D, _D), lambda i: (0, 0))
    b_spec = pl.BlockSpec((1, _D), lambda i: (0, 0))
    return pl.pallas_call(
        _affine_kernel,
        grid=grid,
        in_specs=[
            pl.BlockSpec((1, blk, _D), lambda i: (0, i, 0)),
            pl.BlockSpec((1, blk, _D), lambda i: (1, i, 0)),
            row_spec, row_spec,
            w_spec, w_spec, b_spec,
            w_spec, w_spec, b_spec,
        ],
        out_specs=[row_spec, row_spec],
        out_shape=[
            jax.ShapeDtypeStruct((n, _D), jnp.float32),
            jax.ShapeDtypeStruct((n, _D), jnp.float32),
        ],
    )(agg, agg, x_item, x_user, wm0, wr0, b0.reshape(1, _D),
      wm1, wr1, b1.reshape(1, _D))


def kernel(x_user, x_item, edge_index_u2i, edge_index_i2u,
           W_msg_u2i, W_root_u2i, b_u2i,
           W_msg_i2u, W_root_i2u, b_i2u):
    table = jnp.concatenate([x_user, x_item], axis=0)
    npad_e = _EPAD - _E
    src_pad = jnp.arange(npad_e, dtype=jnp.int32) % _N_USER
    dst_pad = _N_ITEM + (jnp.arange(npad_e, dtype=jnp.int32) % (_N_PAD - _N_ITEM))
    src_flat = jnp.concatenate([
        edge_index_u2i[0].astype(jnp.int32), src_pad,
        edge_index_i2u[0].astype(jnp.int32) + _N_USER, src_pad,
    ])
    dst_flat = jnp.concatenate([
        edge_index_u2i[1].astype(jnp.int32), dst_pad,
        edge_index_i2u[1].astype(jnp.int32), dst_pad,
    ])
    zeros = jnp.zeros((_N_PAD, _D), jnp.float32)
    agg = _sc_aggregate(table, src_flat, dst_flat, zeros)
    out_item, out_user = _tc_epilogue(
        agg, x_item, x_user,
        W_msg_u2i, W_root_u2i, b_u2i,
        W_msg_i2u, W_root_i2u, b_i2u)
    return (out_user, out_item)


# speedup vs baseline: 2.8872x; 1.8451x over previous
import functools

import jax
import jax.numpy as jnp
from jax import lax
from jax.experimental import pallas as pl
from jax.experimental.pallas import tpu as pltpu
from jax.experimental.pallas import tpu_sc as plsc

_N_USER = 10000
_N_ITEM = 10000
_D = 128
_E = 320000

_NUM_TILES = 16
_CHUNK = 80
_EPT = 20480
_NCHUNKS = _EPT // _CHUNK
_EPAD = _EPT * _NUM_TILES
_G = 8
_NPAIRS = _NCHUNKS // 2
_N_PAD = 10240
_ROWS_PER_TILE = _N_PAD // _NUM_TILES


def _sc_aggregate(table, src_flat, dst_flat, zeros):
    mesh = plsc.VectorSubcoreMesh(core_axis_name="c", subcore_axis_name="s")

    @functools.partial(
        pl.kernel,
        out_type=jax.ShapeDtypeStruct((2, _N_PAD, _D), jnp.float32),
        mesh=mesh,
        scratch_types=(
            [pltpu.VMEM((_CHUNK,), jnp.int32)] * 4
            + [pltpu.VMEM((_CHUNK,), jnp.int32)] * 4
            + [pltpu.VMEM((_CHUNK, _D), jnp.float32)] * 2
            + [pltpu.VMEM_SHARED((_N_PAD, _D), jnp.float32)]
            + [pltpu.SemaphoreType.DMA] * 8
        ),
    )
    def agg_kernel(table_hbm, src_hbm, dst_hbm, zeros_hbm, out_hbm, *scr):
        src_v = scr[0:4]
        dst_v = scr[4:8]
        rows_v = scr[8:10]
        acc_sh = scr[10]
        isem = scr[11:15]
        gsem = scr[15:17]
        ssem = scr[17:19]
        c = lax.axis_index("c")
        s = lax.axis_index("s")
        rbase = s * _ROWS_PER_TILE
        ebase = c * _EPAD + s * _EPT

        def idx_copies(i, k):
            e0 = ebase + i * _CHUNK
            return (
                pltpu.make_async_copy(
                    src_hbm.at[pl.ds(e0, _CHUNK)], src_v[k], isem[k]),
                pltpu.make_async_copy(
                    dst_hbm.at[pl.ds(e0, _CHUNK)], dst_v[k], isem[k]),
            )

        def gather(k, p):
            return pltpu.make_async_copy(
                table_hbm.at[src_v[k]], rows_v[p], gsem[p])

        def scatter(k, p):
            return pltpu.make_async_copy(
                rows_v[p], acc_sh.at[dst_v[k]], ssem[p])

        for i0 in range(3):
            for d in idx_copies(i0, i0):
                d.start()
        pltpu.sync_copy(zeros_hbm.at[pl.ds(rbase, _ROWS_PER_TILE)],
                        acc_sh.at[pl.ds(rbase, _ROWS_PER_TILE)])
        plsc.subcore_barrier()
        for d in idx_copies(0, 0):
            d.wait()
        gather(0, 0).start()

        @pl.loop(0, _NCHUNKS // 4)
        def _(t):
            for j in range(4):
                i = t * 4 + j
                p, q = j % 2, (j + 1) % 2
                k1, k3 = (j + 1) % 4, (j + 3) % 4

                gather(j % 4, p).wait()
                scatter(j % 4, p).start(add=True)

                @pl.when(i + 1 < _NCHUNKS)
                def _():
                    for d in idx_copies(i + 1, k1):
                        d.wait()

                @pl.when(i >= 1)
                def _():
                    scatter(k3, q).wait()

                @pl.when(i + 1 < _NCHUNKS)
                def _():
                    gather(k1, q).start()

                @pl.when(i + 3 < _NCHUNKS)
                def _():
                    for d in idx_copies(i + 3, k3):
                        d.start()

        scatter((_NCHUNKS - 1) % 4, (_NCHUNKS - 1) % 2).wait()
        plsc.subcore_barrier()
        pltpu.sync_copy(acc_sh.at[pl.ds(rbase, _ROWS_PER_TILE)],
                        out_hbm.at[c, pl.ds(rbase, _ROWS_PER_TILE)])

    return agg_kernel(table, src_flat, dst_flat, zeros)


def _affine_kernel(agg0_ref, agg1_ref, xi_ref, xu_ref,
                   wm0_ref, wr0_ref, b0_ref, wm1_ref, wr1_ref, b1_ref,
                   oi_ref, ou_ref):
    oi_ref[...] = (
        jnp.dot(agg0_ref[0], wm0_ref[...], preferred_element_type=jnp.float32)
        + jnp.dot(xi_ref[...], wr0_ref[...], preferred_element_type=jnp.float32)
        + b0_ref[...]
    )
    ou_ref[...] = (
        jnp.dot(agg1_ref[0], wm1_ref[...], preferred_element_type=jnp.float32)
        + jnp.dot(xu_ref[...], wr1_ref[...], preferred_element_type=jnp.float32)
        + b1_ref[...]
    )


def _tc_epilogue(agg, x_item, x_user, wm0, wr0, b0, wm1, wr1, b1):
    n = x_item.shape[0]
    blk = 2000
    grid = (n // blk,)
    row_spec = pl.BlockSpec((blk, _D), lambda i: (i, 0))
    w_spec = pl.BlockSpec((_D, _D), lambda i: (0, 0))
    b_spec = pl.BlockSpec((1, _D), lambda i: (0, 0))
    return pl.pallas_call(
        _affine_kernel,
        grid=grid,
        in_specs=[
            pl.BlockSpec((1, blk, _D), lambda i: (0, i, 0)),
            pl.BlockSpec((1, blk, _D), lambda i: (1, i, 0)),
            row_spec, row_spec,
            w_spec, w_spec, b_spec,
            w_spec, w_spec, b_spec,
        ],
        out_specs=[row_spec, row_spec],
        out_shape=[
            jax.ShapeDtypeStruct((n, _D), jnp.float32),
            jax.ShapeDtypeStruct((n, _D), jnp.float32),
        ],
    )(agg, agg, x_item, x_user, wm0, wr0, b0.reshape(1, _D),
      wm1, wr1, b1.reshape(1, _D))


def kernel(x_user, x_item, edge_index_u2i, edge_index_i2u,
           W_msg_u2i, W_root_u2i, b_u2i,
           W_msg_i2u, W_root_i2u, b_i2u):
    table = jnp.concatenate([x_user, x_item], axis=0)
    npad_e = _EPAD - _E
    src_pad = jnp.arange(npad_e, dtype=jnp.int32) % _N_USER
    dst_pad = _N_ITEM + (jnp.arange(npad_e, dtype=jnp.int32) % (_N_PAD - _N_ITEM))
    src_flat = jnp.concatenate([
        edge_index_u2i[0].astype(jnp.int32), src_pad,
        edge_index_i2u[0].astype(jnp.int32) + _N_USER, src_pad,
    ])
    dst_flat = jnp.concatenate([
        edge_index_u2i[1].astype(jnp.int32), dst_pad,
        edge_index_i2u[1].astype(jnp.int32), dst_pad,
    ])
    zeros = jnp.zeros((_N_PAD, _D), jnp.float32)
    agg = _sc_aggregate(table, src_flat, dst_flat, zeros)
    out_item, out_user = _tc_epilogue(
        agg, x_item, x_user,
        W_msg_u2i, W_root_u2i, b_u2i,
        W_msg_i2u, W_root_i2u, b_i2u)
    return (out_user, out_item)


# speedup vs baseline: 3.4181x; 1.1839x over previous
import functools

import jax
import jax.numpy as jnp
from jax import lax
from jax.experimental import pallas as pl
from jax.experimental.pallas import tpu as pltpu
from jax.experimental.pallas import tpu_sc as plsc

_N_USER = 10000
_N_ITEM = 10000
_D = 128
_E = 320000

_NUM_TILES = 16
_CHUNK = 128
_EPT = 20480
_NCHUNKS = _EPT // _CHUNK
_EPAD = _EPT * _NUM_TILES
_G = 8
_NPAIRS = _NCHUNKS // 2
_N_PAD = 10240
_ROWS_PER_TILE = _N_PAD // _NUM_TILES


def _sc_aggregate(table, src_flat, dst_flat, zeros):
    mesh = plsc.VectorSubcoreMesh(core_axis_name="c", subcore_axis_name="s")

    @functools.partial(
        pl.kernel,
        out_type=jax.ShapeDtypeStruct((2, _N_PAD, _D), jnp.float32),
        mesh=mesh,
        scratch_types=(
            [pltpu.VMEM((_CHUNK,), jnp.int32)] * 4
            + [pltpu.VMEM((_CHUNK,), jnp.int32)] * 4
            + [pltpu.VMEM((_CHUNK, _D), jnp.float32)] * 2
            + [pltpu.VMEM_SHARED((_N_PAD, _D), jnp.float32)]
            + [pltpu.SemaphoreType.DMA] * 8
        ),
    )
    def agg_kernel(table_hbm, src_hbm, dst_hbm, zeros_hbm, out_hbm, *scr):
        src_v = scr[0:4]
        dst_v = scr[4:8]
        rows_v = scr[8:10]
        acc_sh = scr[10]
        isem = scr[11:15]
        gsem = scr[15:17]
        ssem = scr[17:19]
        c = lax.axis_index("c")
        s = lax.axis_index("s")
        rbase = s * _ROWS_PER_TILE
        ebase = c * _EPAD + s * _EPT

        def idx_copies(i, k):
            e0 = ebase + i * _CHUNK
            return (
                pltpu.make_async_copy(
                    src_hbm.at[pl.ds(e0, _CHUNK)], src_v[k], isem[k]),
                pltpu.make_async_copy(
                    dst_hbm.at[pl.ds(e0, _CHUNK)], dst_v[k], isem[k]),
            )

        def gather(k, p):
            return pltpu.make_async_copy(
                table_hbm.at[src_v[k]], rows_v[p], gsem[p])

        def scatter(k, p):
            return pltpu.make_async_copy(
                rows_v[p], acc_sh.at[dst_v[k]], ssem[p])

        for i0 in range(3):
            for d in idx_copies(i0, i0):
                d.start()
        pltpu.sync_copy(zeros_hbm.at[pl.ds(rbase, _ROWS_PER_TILE)],
                        acc_sh.at[pl.ds(rbase, _ROWS_PER_TILE)])
        plsc.subcore_barrier()
        for d in idx_copies(0, 0):
            d.wait()
        gather(0, 0).start()

        @pl.loop(0, _NCHUNKS // 4)
        def _(t):
            for j in range(4):
                i = t * 4 + j
                p, q = j % 2, (j + 1) % 2
                k1, k3 = (j + 1) % 4, (j + 3) % 4

                gather(j % 4, p).wait()
                scatter(j % 4, p).start(add=True)

                @pl.when(i + 1 < _NCHUNKS)
                def _():
                    for d in idx_copies(i + 1, k1):
                        d.wait()

                @pl.when(i >= 1)
                def _():
                    scatter(k3, q).wait()

                @pl.when(i + 1 < _NCHUNKS)
                def _():
                    gather(k1, q).start()

                @pl.when(i + 3 < _NCHUNKS)
                def _():
                    for d in idx_copies(i + 3, k3):
                        d.start()

        scatter((_NCHUNKS - 1) % 4, (_NCHUNKS - 1) % 2).wait()
        plsc.subcore_barrier()
        pltpu.sync_copy(acc_sh.at[pl.ds(rbase, _ROWS_PER_TILE)],
                        out_hbm.at[c, pl.ds(rbase, _ROWS_PER_TILE)])

    return agg_kernel(table, src_flat, dst_flat, zeros)


def _affine_kernel(agg0_ref, agg1_ref, xi_ref, xu_ref,
                   wm0_ref, wr0_ref, b0_ref, wm1_ref, wr1_ref, b1_ref,
                   oi_ref, ou_ref):
    oi_ref[...] = (
        jnp.dot(agg0_ref[0], wm0_ref[...], preferred_element_type=jnp.float32)
        + jnp.dot(xi_ref[...], wr0_ref[...], preferred_element_type=jnp.float32)
        + b0_ref[...]
    )
    ou_ref[...] = (
        jnp.dot(agg1_ref[0], wm1_ref[...], preferred_element_type=jnp.float32)
        + jnp.dot(xu_ref[...], wr1_ref[...], preferred_element_type=jnp.float32)
        + b1_ref[...]
    )


def _tc_epilogue(agg, x_item, x_user, wm0, wr0, b0, wm1, wr1, b1):
    n = x_item.shape[0]
    blk = 2000
    grid = (n // blk,)
    row_spec = pl.BlockSpec((blk, _D), lambda i: (i, 0))
    w_spec = pl.BlockSpec((_D, _D), lambda i: (0, 0))
    b_spec = pl.BlockSpec((1, _D), lambda i: (0, 0))
    return pl.pallas_call(
        _affine_kernel,
        grid=grid,
        in_specs=[
            pl.BlockSpec((1, blk, _D), lambda i: (0, i, 0)),
            pl.BlockSpec((1, blk, _D), lambda i: (1, i, 0)),
            row_spec, row_spec,
            w_spec, w_spec, b_spec,
            w_spec, w_spec, b_spec,
        ],
        out_specs=[row_spec, row_spec],
        out_shape=[
            jax.ShapeDtypeStruct((n, _D), jnp.float32),
            jax.ShapeDtypeStruct((n, _D), jnp.float32),
        ],
    )(agg, agg, x_item, x_user, wm0, wr0, b0.reshape(1, _D),
      wm1, wr1, b1.reshape(1, _D))


def kernel(x_user, x_item, edge_index_u2i, edge_index_i2u,
           W_msg_u2i, W_root_u2i, b_u2i,
           W_msg_i2u, W_root_i2u, b_i2u):
    table = jnp.concatenate([x_user, x_item], axis=0)
    npad_e = _EPAD - _E
    src_pad = jnp.arange(npad_e, dtype=jnp.int32) % _N_USER
    dst_pad = _N_ITEM + (jnp.arange(npad_e, dtype=jnp.int32) % (_N_PAD - _N_ITEM))
    src_flat = jnp.concatenate([
        edge_index_u2i[0].astype(jnp.int32), src_pad,
        edge_index_i2u[0].astype(jnp.int32) + _N_USER, src_pad,
    ])
    dst_flat = jnp.concatenate([
        edge_index_u2i[1].astype(jnp.int32), dst_pad,
        edge_index_i2u[1].astype(jnp.int32), dst_pad,
    ])
    zeros = jnp.zeros((_N_PAD, _D), jnp.float32)
    agg = _sc_aggregate(table, src_flat, dst_flat, zeros)
    out_item, out_user = _tc_epilogue(
        agg, x_item, x_user,
        W_msg_u2i, W_root_u2i, b_u2i,
        W_msg_i2u, W_root_i2u, b_i2u)
    return (out_user, out_item)


# speedup vs baseline: 4.3534x; 1.2736x over previous
import functools

import jax
import jax.numpy as jnp
from jax import lax
from jax.experimental import pallas as pl
from jax.experimental.pallas import tpu as pltpu
from jax.experimental.pallas import tpu_sc as plsc

_N_USER = 10000
_N_ITEM = 10000
_D = 128
_E = 320000

_NUM_TILES = 16
_CHUNK = 120
_EPT = 20160
_NCHUNKS = _EPT // _CHUNK
_EPAD = _EPT * _NUM_TILES
_N_PAD = 10112
_ROWS_PER_TILE = _N_PAD // _NUM_TILES


def _sc_aggregate(table, src_flat, dst_flat, zeros):
    mesh = plsc.VectorSubcoreMesh(core_axis_name="c", subcore_axis_name="s")

    @functools.partial(
        pl.kernel,
        out_type=jax.ShapeDtypeStruct((2, _N_PAD, _D), jnp.float32),
        mesh=mesh,
        scratch_types=(
            [pltpu.VMEM((_CHUNK,), jnp.int32)] * 4
            + [pltpu.VMEM((_CHUNK,), jnp.int32)] * 4
            + [pltpu.VMEM((_CHUNK, _D), jnp.float32)] * 3
            + [pltpu.VMEM_SHARED((_N_PAD, _D), jnp.float32)]
            + [pltpu.SemaphoreType.DMA] * 10
        ),
    )
    def agg_kernel(table_hbm, src_hbm, dst_hbm, zeros_hbm, out_hbm, *scr):
        src_v = scr[0:4]
        dst_v = scr[4:8]
        rows_v = scr[8:11]
        acc_sh = scr[11]
        isem = scr[12:16]
        gsem = scr[16:19]
        ssem = scr[19:22]
        c = lax.axis_index("c")
        s = lax.axis_index("s")
        rbase = s * _ROWS_PER_TILE
        ebase = c * _EPAD + s * _EPT

        def idx_copies(i, k):
            e0 = ebase + i * _CHUNK
            return (
                pltpu.make_async_copy(
                    src_hbm.at[pl.ds(e0, _CHUNK)], src_v[k], isem[k]),
                pltpu.make_async_copy(
                    dst_hbm.at[pl.ds(e0, _CHUNK)], dst_v[k], isem[k]),
            )

        def gather(k, p):
            return pltpu.make_async_copy(
                table_hbm.at[src_v[k]], rows_v[p], gsem[p])

        def scatter(k, p):
            return pltpu.make_async_copy(
                rows_v[p], acc_sh.at[dst_v[k]], ssem[p])

        for i0 in range(3):
            for d in idx_copies(i0, i0):
                d.start()
        pltpu.sync_copy(zeros_hbm.at[pl.ds(rbase, _ROWS_PER_TILE)],
                        acc_sh.at[pl.ds(rbase, _ROWS_PER_TILE)])
        plsc.subcore_barrier()
        for d in idx_copies(0, 0):
            d.wait()
        gather(0, 0).start()
        for d in idx_copies(1, 1):
            d.wait()
        gather(1, 1).start()

        @pl.loop(0, _NCHUNKS // 12)
        def _(t):
            for j in range(12):
                i = t * 12 + j
                p, p2 = j % 3, (j + 2) % 3
                k2, k3 = (j + 2) % 4, (j + 3) % 4

                gather(j % 4, p).wait()
                scatter(j % 4, p).start(add=True)

                @pl.when(i >= 1)
                def _():
                    scatter(k3, p2).wait()

                @pl.when(i + 2 < _NCHUNKS)
                def _():
                    for d in idx_copies(i + 2, k2):
                        d.wait()
                    gather(k2, p2).start()

                @pl.when(i + 3 < _NCHUNKS)
                def _():
                    for d in idx_copies(i + 3, k3):
                        d.start()

        scatter((_NCHUNKS - 1) % 4, (_NCHUNKS - 1) % 3).wait()
        plsc.subcore_barrier()
        pltpu.sync_copy(acc_sh.at[pl.ds(rbase, _ROWS_PER_TILE)],
                        out_hbm.at[c, pl.ds(rbase, _ROWS_PER_TILE)])

    return agg_kernel(table, src_flat, dst_flat, zeros)


def _affine_kernel(agg0_ref, agg1_ref, xi_ref, xu_ref,
                   wm0_ref, wr0_ref, b0_ref, wm1_ref, wr1_ref, b1_ref,
                   oi_ref, ou_ref):
    oi_ref[...] = (
        jnp.dot(agg0_ref[0], wm0_ref[...], preferred_element_type=jnp.float32)
        + jnp.dot(xi_ref[...], wr0_ref[...], preferred_element_type=jnp.float32)
        + b0_ref[...]
    )
    ou_ref[...] = (
        jnp.dot(agg1_ref[0], wm1_ref[...], preferred_element_type=jnp.float32)
        + jnp.dot(xu_ref[...], wr1_ref[...], preferred_element_type=jnp.float32)
        + b1_ref[...]
    )


def _tc_epilogue(agg, x_item, x_user, wm0, wr0, b0, wm1, wr1, b1):
    n = x_item.shape[0]
    blk = 2000
    grid = (n // blk,)
    row_spec = pl.BlockSpec((blk, _D), lambda i: (i, 0))
    w_spec = pl.BlockSpec((_D, _D), lambda i: (0, 0))
    b_spec = pl.BlockSpec((1, _D), lambda i: (0, 0))
    return pl.pallas_call(
        _affine_kernel,
        grid=grid,
        in_specs=[
            pl.BlockSpec((1, blk, _D), lambda i: (0, i, 0)),
            pl.BlockSpec((1, blk, _D), lambda i: (1, i, 0)),
            row_spec, row_spec,
            w_spec, w_spec, b_spec,
            w_spec, w_spec, b_spec,
        ],
        out_specs=[row_spec, row_spec],
        out_shape=[
            jax.ShapeDtypeStruct((n, _D), jnp.float32),
            jax.ShapeDtypeStruct((n, _D), jnp.float32),
        ],
    )(agg, agg, x_item, x_user, wm0, wr0, b0.reshape(1, _D),
      wm1, wr1, b1.reshape(1, _D))


def kernel(x_user, x_item, edge_index_u2i, edge_index_i2u,
           W_msg_u2i, W_root_u2i, b_u2i,
           W_msg_i2u, W_root_i2u, b_i2u):
    table = jnp.concatenate([x_user, x_item], axis=0)
    npad_e = _EPAD - _E
    src_pad = jnp.arange(npad_e, dtype=jnp.int32) % _N_USER
    dst_pad = _N_ITEM + (jnp.arange(npad_e, dtype=jnp.int32) % (_N_PAD - _N_ITEM))
    src_flat = jnp.concatenate([
        edge_index_u2i[0].astype(jnp.int32), src_pad,
        edge_index_i2u[0].astype(jnp.int32) + _N_USER, src_pad,
    ])
    dst_flat = jnp.concatenate([
        edge_index_u2i[1].astype(jnp.int32), dst_pad,
        edge_index_i2u[1].astype(jnp.int32), dst_pad,
    ])
    zeros = jnp.zeros((_N_PAD, _D), jnp.float32)
    agg = _sc_aggregate(table, src_flat, dst_flat, zeros)
    out_item, out_user = _tc_epilogue(
        agg, x_item, x_user,
        W_msg_u2i, W_root_u2i, b_u2i,
        W_msg_i2u, W_root_i2u, b_i2u)
    return (out_user, out_item)
